# baseline (speedup 1.0000x reference)
import functools

import jax
import jax.numpy as jnp
from jax.experimental import pallas as pl
from jax.experimental.pallas import tpu as pltpu
from jax.experimental.pallas import tpu_sc as plsc

E = 8
K = 2
C = 768
H = 3072
T = 2048
BLK = 256
NBLK = (T * K + E * (BLK - 1) + BLK - 1) // BLK
P = NBLK * BLK
NC = 2
NS = 16
NW = NC * NS
TPW = T // NW


def _lane_cumsum(x):
    c = x
    sh = 1
    n = x.shape[1]
    while sh < n:
        z = jnp.zeros((x.shape[0], sh), c.dtype)
        c = c + jnp.concatenate([z, c[:, :n - sh]], axis=1)
        sh *= 2
    return c


def _router_body(zt_ref, d0_ref, d1_ref, v0_ref, v1_ref, blk_e_ref, tp_ref,
                 loss_ref):
    zt = zt_ref[...]
    m = jnp.max(zt, axis=0, keepdims=True)
    ez = jnp.exp(zt - m)
    gates = ez / jnp.sum(ez, axis=0, keepdims=True)

    eidx = jax.lax.broadcasted_iota(jnp.int32, (E, T), 0)
    e0 = jnp.argmax(zt, axis=0)[None, :]
    is0 = eidx == e0
    v0_ref[...] = jnp.sum(jnp.where(is0, gates, 0.0), axis=0, keepdims=True)
    z1 = jnp.where(is0, -jnp.inf, zt)
    e1 = jnp.argmax(z1, axis=0)[None, :]
    is1 = eidx == e1
    v1_ref[...] = jnp.sum(jnp.where(is1, gates, 0.0), axis=0, keepdims=True)

    mg = jnp.sum(gates, axis=1, keepdims=True) / T
    loss_ref[...] = jnp.sum(mg * jnp.log(mg + 1e-8), axis=0, keepdims=True)

    oh0 = is0.astype(jnp.int32)
    oh1 = is1.astype(jnp.int32)
    c0 = _lane_cumsum(oh0)
    c1 = _lane_cumsum(oh1)
    tot0 = c0[:, T - 1:T]
    cnt = tot0 + c1[:, T - 1:T]
    padded = ((cnt + BLK - 1) // BLK) * BLK
    tri = (jax.lax.broadcasted_iota(jnp.int32, (E, E), 0) >
           jax.lax.broadcasted_iota(jnp.int32, (E, E), 1)).astype(jnp.float32)
    seg = jax.lax.dot_general(
        tri, padded.astype(jnp.float32), (((1,), (0,)), ((), ())),
        preferred_element_type=jnp.float32).astype(jnp.int32)
    d0_ref[...] = jnp.sum(
        jnp.where(is0, seg + c0 - oh0, 0), axis=0, keepdims=True)
    d1_ref[...] = jnp.sum(
        jnp.where(is1, seg + tot0 + c1 - oh1, 0), axis=0, keepdims=True)

    total_padded = jnp.sum(padded)
    tp_ref[...] = total_padded[None, None]
    bstart = jax.lax.broadcasted_iota(jnp.int32, (1, NBLK), 1) * BLK
    bstart = jnp.minimum(bstart, total_padded - BLK)
    blk_e_ref[...] = jnp.sum(
        (seg <= bstart).astype(jnp.int32), axis=0, keepdims=True) - 1


def _ffn_body(blk_e_ref, tp_ref, xbf_ref, d0_ref, d1_ref, v0_ref, v1_ref,
              w1_ref, b1_ref, w2_ref, b2_ref, y_ref, w1c_ref, w2c_ref):
    i = pl.program_id(0)
    prev = blk_e_ref[jnp.maximum(i - 1, 0)]

    @pl.when((i == 0) | (blk_e_ref[i] != prev))
    def _recast():
        w1c_ref[...] = w1_ref[0].astype(jnp.bfloat16)
        w2c_ref[...] = w2_ref[0].astype(jnp.bfloat16)

    @pl.when(i * BLK < tp_ref[0])
    def _compute():
        rowid = i * BLK + jax.lax.broadcasted_iota(jnp.int32, (BLK, T), 0)
        m0 = d0_ref[...] == rowid
        m1 = d1_ref[...] == rowid
        onehot = jnp.where(m0 | m1, 1.0, 0.0).astype(jnp.bfloat16)
        pwcol = jnp.sum(jnp.where(m0, v0_ref[...], 0.0) +
                        jnp.where(m1, v1_ref[...], 0.0), axis=1,
                        keepdims=True)
        xb = jax.lax.dot_general(
            onehot, xbf_ref[...], (((1,), (0,)), ((), ())),
            preferred_element_type=jnp.float32).astype(jnp.bfloat16)
        h = jax.lax.dot_general(
            xb, w1c_ref[...], (((1,), (1,)), ((), ())),
            preferred_element_type=jnp.float32) + b1_ref[0]
        h = (h * 0.5 * (1.0 + jax.lax.erf(h * (2.0 ** -0.5)))).astype(
            jnp.bfloat16)
        y = jax.lax.dot_general(
            h, w2c_ref[...], (((1,), (1,)), ((), ())),
            preferred_element_type=jnp.float32) + b2_ref[0]
        y_ref[...] = y * pwcol

    @pl.when(i * BLK >= tp_ref[0])
    def _zero():
        y_ref[...] = jnp.zeros_like(y_ref)


CH = 16
NCHUNK = TPW // CH


def _sc_combine_body(y_hbm, d0_hbm, d1_hbm, out_hbm, idx0, idx1, rows0, rows1,
                     sem0, sem1):
    wid = jax.lax.axis_index("s") * NC + jax.lax.axis_index("c")
    base = wid * TPW
    pltpu.sync_copy(d0_hbm.at[pl.ds(base, TPW)], idx0)
    pltpu.sync_copy(d1_hbm.at[pl.ds(base, TPW)], idx1)
    sems = (sem0, sem1)

    def issue(c, b):
        i0 = idx0[pl.ds(c * CH, CH)]
        i1 = idx1[pl.ds(c * CH, CH)]
        cp0 = pltpu.async_copy(y_hbm.at[i0], rows0.at[b], sems[b])
        cp1 = pltpu.async_copy(y_hbm.at[i1], rows1.at[b], sems[b])
        return cp0, cp1

    cps = [None, None]
    cps[0] = issue(0, 0)
    for c in range(NCHUNK):
        b = c % 2
        if c + 1 < NCHUNK:
            cps[1 - b] = issue(c + 1, 1 - b)
        cps[b][0].wait()
        cps[b][1].wait()

        def row_body(t, carry):
            for j in range(C // 16):
                sl = pl.ds(j * 16, 16)
                rows0[b, t, sl] = rows0[b, t, sl] + rows1[b, t, sl]
            return carry

        jax.lax.fori_loop(0, CH, row_body, 0)
        pltpu.sync_copy(rows0.at[b], out_hbm.at[pl.ds(base + c * CH, CH)])


def kernel(x, gate_w, gate_b, w1, b1, w2, b2):
    xf = x.reshape(-1, C)

    logits = xf @ gate_w.T + gate_b
    u = jax.random.uniform(jax.random.key(42), logits.shape,
                           minval=1e-9, maxval=1.0, dtype=jnp.float32)
    z = logits - jnp.log(-jnp.log(u))

    d0, d1, v0, v1, blk_e2, tp2, lossv = pl.pallas_call(
        _router_body,
        grid=(1,),
        in_specs=[pl.BlockSpec((E, T), lambda i: (0, 0))],
        out_specs=[
            pl.BlockSpec((1, T), lambda i: (0, 0)),
            pl.BlockSpec((1, T), lambda i: (0, 0)),
            pl.BlockSpec((1, T), lambda i: (0, 0)),
            pl.BlockSpec((1, T), lambda i: (0, 0)),
            pl.BlockSpec((1, NBLK), lambda i: (0, 0)),
            pl.BlockSpec((1, 1), lambda i: (0, 0)),
            pl.BlockSpec((1, 1), lambda i: (0, 0)),
        ],
        out_shape=[
            jax.ShapeDtypeStruct((1, T), jnp.int32),
            jax.ShapeDtypeStruct((1, T), jnp.int32),
            jax.ShapeDtypeStruct((1, T), jnp.float32),
            jax.ShapeDtypeStruct((1, T), jnp.float32),
            jax.ShapeDtypeStruct((1, NBLK), jnp.int32),
            jax.ShapeDtypeStruct((1, 1), jnp.int32),
            jax.ShapeDtypeStruct((1, 1), jnp.float32),
        ],
    )(z.T)
    loss = lossv.reshape(())

    y = pl.pallas_call(
        _ffn_body,
        grid_spec=pltpu.PrefetchScalarGridSpec(
            num_scalar_prefetch=2,
            grid=(NBLK,),
            in_specs=[
                pl.BlockSpec((T, C), lambda i, be, tp: (0, 0)),
                pl.BlockSpec((1, T), lambda i, be, tp: (0, 0)),
                pl.BlockSpec((1, T), lambda i, be, tp: (0, 0)),
                pl.BlockSpec((1, T), lambda i, be, tp: (0, 0)),
                pl.BlockSpec((1, T), lambda i, be, tp: (0, 0)),
                pl.BlockSpec((1, H, C), lambda i, be, tp: (be[i], 0, 0)),
                pl.BlockSpec((1, 1, H), lambda i, be, tp: (be[i], 0, 0)),
                pl.BlockSpec((1, C, H), lambda i, be, tp: (be[i], 0, 0)),
                pl.BlockSpec((1, 1, C), lambda i, be, tp: (be[i], 0, 0)),
            ],
            out_specs=pl.BlockSpec((BLK, C), lambda i, be, tp: (i, 0)),
            scratch_shapes=[
                pltpu.VMEM((H, C), jnp.bfloat16),
                pltpu.VMEM((C, H), jnp.bfloat16),
            ],
        ),
        out_shape=jax.ShapeDtypeStruct((P, C), jnp.float32),
        compiler_params=pltpu.CompilerParams(
            dimension_semantics=("arbitrary",)),
    )(blk_e2.reshape(NBLK), tp2.reshape(1), xf.astype(jnp.bfloat16),
      d0, d1, v0, v1, w1, b1.reshape(E, 1, H), w2, b2.reshape(E, 1, C))

    combine = functools.partial(
        pl.kernel,
        out_type=jax.ShapeDtypeStruct((T, C), jnp.float32),
        mesh=plsc.VectorSubcoreMesh(core_axis_name="c", subcore_axis_name="s",
                                    num_cores=NC, num_subcores=NS),
        scratch_types=[
            pltpu.VMEM((TPW,), jnp.int32),
            pltpu.VMEM((TPW,), jnp.int32),
            pltpu.VMEM((2, CH, C), jnp.float32),
            pltpu.VMEM((2, CH, C), jnp.float32),
            pltpu.SemaphoreType.DMA,
            pltpu.SemaphoreType.DMA,
        ],
    )(_sc_combine_body)
    out = combine(y, d0.reshape(T), d1.reshape(T))

    return out.reshape(x.shape), loss


# speedup vs baseline: 1.0291x; 1.0291x over previous
import functools

import jax
import jax.numpy as jnp
from jax.experimental import pallas as pl
from jax.experimental.pallas import tpu as pltpu
from jax.experimental.pallas import tpu_sc as plsc

E = 8
K = 2
C = 768
H = 3072
T = 2048
BLK = 512
NBLK = (T * K + E * (BLK - 1) + BLK - 1) // BLK
P = NBLK * BLK
NC = 2
NS = 16
NW = NC * NS
TPW = T // NW


def _lane_cumsum(x):
    c = x
    sh = 1
    n = x.shape[1]
    while sh < n:
        z = jnp.zeros((x.shape[0], sh), c.dtype)
        c = c + jnp.concatenate([z, c[:, :n - sh]], axis=1)
        sh *= 2
    return c


def _router_body(zt_ref, d0_ref, d1_ref, v0_ref, v1_ref, blk_e_ref, tp_ref,
                 loss_ref):
    zt = zt_ref[...]
    m = jnp.max(zt, axis=0, keepdims=True)
    ez = jnp.exp(zt - m)
    gates = ez / jnp.sum(ez, axis=0, keepdims=True)

    eidx = jax.lax.broadcasted_iota(jnp.int32, (E, T), 0)
    e0 = jnp.argmax(zt, axis=0)[None, :]
    is0 = eidx == e0
    v0_ref[...] = jnp.sum(jnp.where(is0, gates, 0.0), axis=0, keepdims=True)
    z1 = jnp.where(is0, -jnp.inf, zt)
    e1 = jnp.argmax(z1, axis=0)[None, :]
    is1 = eidx == e1
    v1_ref[...] = jnp.sum(jnp.where(is1, gates, 0.0), axis=0, keepdims=True)

    mg = jnp.sum(gates, axis=1, keepdims=True) / T
    loss_ref[...] = jnp.sum(mg * jnp.log(mg + 1e-8), axis=0, keepdims=True)

    oh0 = is0.astype(jnp.int32)
    oh1 = is1.astype(jnp.int32)
    c0 = _lane_cumsum(oh0)
    c1 = _lane_cumsum(oh1)
    tot0 = c0[:, T - 1:T]
    cnt = tot0 + c1[:, T - 1:T]
    padded = ((cnt + BLK - 1) // BLK) * BLK
    tri = (jax.lax.broadcasted_iota(jnp.int32, (E, E), 0) >
           jax.lax.broadcasted_iota(jnp.int32, (E, E), 1)).astype(jnp.float32)
    seg = jax.lax.dot_general(
        tri, padded.astype(jnp.float32), (((1,), (0,)), ((), ())),
        preferred_element_type=jnp.float32).astype(jnp.int32)
    d0_ref[...] = jnp.sum(
        jnp.where(is0, seg + c0 - oh0, 0), axis=0, keepdims=True)
    d1_ref[...] = jnp.sum(
        jnp.where(is1, seg + tot0 + c1 - oh1, 0), axis=0, keepdims=True)

    total_padded = jnp.sum(padded)
    tp_ref[...] = total_padded[None, None]
    bstart = jax.lax.broadcasted_iota(jnp.int32, (1, NBLK), 1) * BLK
    bstart = jnp.minimum(bstart, total_padded - BLK)
    blk_e_ref[...] = jnp.sum(
        (seg <= bstart).astype(jnp.int32), axis=0, keepdims=True) - 1


def _ffn_body(blk_e_ref, tp_ref, xbf_ref, d0_ref, d1_ref, v0_ref, v1_ref,
              w1_ref, b1_ref, w2_ref, b2_ref, y_ref):
    i = pl.program_id(0)

    @pl.when(i * BLK < tp_ref[0])
    def _compute():
        rowid = i * BLK + jax.lax.broadcasted_iota(jnp.int32, (BLK, T), 0)
        m0 = d0_ref[...] == rowid
        m1 = d1_ref[...] == rowid
        onehot = jnp.where(m0 | m1, 1.0, 0.0).astype(jnp.bfloat16)
        pwcol = jnp.sum(jnp.where(m0, v0_ref[...], 0.0) +
                        jnp.where(m1, v1_ref[...], 0.0), axis=1,
                        keepdims=True)
        xb = jax.lax.dot_general(
            onehot, xbf_ref[...], (((1,), (0,)), ((), ())),
            preferred_element_type=jnp.float32).astype(jnp.bfloat16)
        h = jax.lax.dot_general(
            xb, w1_ref[0].astype(jnp.bfloat16), (((1,), (1,)), ((), ())),
            preferred_element_type=jnp.float32) + b1_ref[0]
        h = (h * 0.5 * (1.0 + jax.lax.erf(h * (2.0 ** -0.5)))).astype(
            jnp.bfloat16)
        y = jax.lax.dot_general(
            h, w2_ref[0].astype(jnp.bfloat16), (((1,), (1,)), ((), ())),
            preferred_element_type=jnp.float32) + b2_ref[0]
        y_ref[...] = y * pwcol

    @pl.when(i * BLK >= tp_ref[0])
    def _zero():
        y_ref[...] = jnp.zeros_like(y_ref)


CH = 16
NCHUNK = TPW // CH


def _sc_combine_body(y_hbm, d0_hbm, d1_hbm, out_hbm, idx0, idx1, rows0, rows1,
                     sem0, sem1):
    wid = jax.lax.axis_index("s") * NC + jax.lax.axis_index("c")
    base = wid * TPW
    pltpu.sync_copy(d0_hbm.at[pl.ds(base, TPW)], idx0)
    pltpu.sync_copy(d1_hbm.at[pl.ds(base, TPW)], idx1)
    sems = (sem0, sem1)

    def issue(c, b):
        i0 = idx0[pl.ds(c * CH, CH)]
        i1 = idx1[pl.ds(c * CH, CH)]
        cp0 = pltpu.async_copy(y_hbm.at[i0], rows0.at[b], sems[b])
        cp1 = pltpu.async_copy(y_hbm.at[i1], rows1.at[b], sems[b])
        return cp0, cp1

    cps = [None, None]
    cps[0] = issue(0, 0)
    for c in range(NCHUNK):
        b = c % 2
        if c + 1 < NCHUNK:
            cps[1 - b] = issue(c + 1, 1 - b)
        cps[b][0].wait()
        cps[b][1].wait()

        def row_body(t, carry):
            for j in range(C // 16):
                sl = pl.ds(j * 16, 16)
                rows0[b, t, sl] = rows0[b, t, sl] + rows1[b, t, sl]
            return carry

        jax.lax.fori_loop(0, CH, row_body, 0)
        pltpu.sync_copy(rows0.at[b], out_hbm.at[pl.ds(base + c * CH, CH)])


def kernel(x, gate_w, gate_b, w1, b1, w2, b2):
    xf = x.reshape(-1, C)

    logits = xf @ gate_w.T + gate_b
    u = jax.random.uniform(jax.random.key(42), logits.shape,
                           minval=1e-9, maxval=1.0, dtype=jnp.float32)
    z = logits - jnp.log(-jnp.log(u))

    d0, d1, v0, v1, blk_e2, tp2, lossv = pl.pallas_call(
        _router_body,
        grid=(1,),
        in_specs=[pl.BlockSpec((E, T), lambda i: (0, 0))],
        out_specs=[
            pl.BlockSpec((1, T), lambda i: (0, 0)),
            pl.BlockSpec((1, T), lambda i: (0, 0)),
            pl.BlockSpec((1, T), lambda i: (0, 0)),
            pl.BlockSpec((1, T), lambda i: (0, 0)),
            pl.BlockSpec((1, NBLK), lambda i: (0, 0)),
            pl.BlockSpec((1, 1), lambda i: (0, 0)),
            pl.BlockSpec((1, 1), lambda i: (0, 0)),
        ],
        out_shape=[
            jax.ShapeDtypeStruct((1, T), jnp.int32),
            jax.ShapeDtypeStruct((1, T), jnp.int32),
            jax.ShapeDtypeStruct((1, T), jnp.float32),
            jax.ShapeDtypeStruct((1, T), jnp.float32),
            jax.ShapeDtypeStruct((1, NBLK), jnp.int32),
            jax.ShapeDtypeStruct((1, 1), jnp.int32),
            jax.ShapeDtypeStruct((1, 1), jnp.float32),
        ],
    )(z.T)
    loss = lossv.reshape(())

    y = pl.pallas_call(
        _ffn_body,
        grid_spec=pltpu.PrefetchScalarGridSpec(
            num_scalar_prefetch=2,
            grid=(NBLK,),
            in_specs=[
                pl.BlockSpec((T, C), lambda i, be, tp: (0, 0)),
                pl.BlockSpec((1, T), lambda i, be, tp: (0, 0)),
                pl.BlockSpec((1, T), lambda i, be, tp: (0, 0)),
                pl.BlockSpec((1, T), lambda i, be, tp: (0, 0)),
                pl.BlockSpec((1, T), lambda i, be, tp: (0, 0)),
                pl.BlockSpec((1, H, C), lambda i, be, tp: (be[i], 0, 0)),
                pl.BlockSpec((1, 1, H), lambda i, be, tp: (be[i], 0, 0)),
                pl.BlockSpec((1, C, H), lambda i, be, tp: (be[i], 0, 0)),
                pl.BlockSpec((1, 1, C), lambda i, be, tp: (be[i], 0, 0)),
            ],
            out_specs=pl.BlockSpec((BLK, C), lambda i, be, tp: (i, 0)),
        ),
        out_shape=jax.ShapeDtypeStruct((P, C), jnp.float32),
        compiler_params=pltpu.CompilerParams(
            dimension_semantics=("arbitrary",)),
    )(blk_e2.reshape(NBLK), tp2.reshape(1), xf.astype(jnp.bfloat16),
      d0, d1, v0, v1, w1, b1.reshape(E, 1, H), w2, b2.reshape(E, 1, C))

    combine = functools.partial(
        pl.kernel,
        out_type=jax.ShapeDtypeStruct((T, C), jnp.float32),
        mesh=plsc.VectorSubcoreMesh(core_axis_name="c", subcore_axis_name="s",
                                    num_cores=NC, num_subcores=NS),
        scratch_types=[
            pltpu.VMEM((TPW,), jnp.int32),
            pltpu.VMEM((TPW,), jnp.int32),
            pltpu.VMEM((2, CH, C), jnp.float32),
            pltpu.VMEM((2, CH, C), jnp.float32),
            pltpu.SemaphoreType.DMA,
            pltpu.SemaphoreType.DMA,
        ],
    )(_sc_combine_body)
    out = combine(y, d0.reshape(T), d1.reshape(T))

    return out.reshape(x.shape), loss
